# baseline (speedup 1.0000x reference)
import functools
import math

import jax
import jax.numpy as jnp
from jax import lax
from jax.experimental import pallas as pl
from jax.experimental.pallas import tpu as pltpu
from jax.experimental.pallas import tpu_sc as plsc

EMBED_DIM = 64
SCALE = math.sqrt(EMBED_DIM)

NUM_CORES = 2
NUM_SUBCORES = 16
NW = NUM_CORES * NUM_SUBCORES

IDX_SUB = 128
SUBS_PER_CHUNK = 8
CHUNK = IDX_SUB * SUBS_PER_CHUNK


def _make_kernel(n_rows: int):
    b_per_w = n_rows // NW
    n_chunks = b_per_w // CHUNK

    mesh = plsc.VectorSubcoreMesh(core_axis_name="c", subcore_axis_name="s")

    @functools.partial(
        pl.kernel,
        mesh=mesh,
        out_type=jax.ShapeDtypeStruct((n_rows, EMBED_DIM), jnp.float32),
        scratch_types=[
            pltpu.VMEM((SUBS_PER_CHUNK, IDX_SUB), jnp.int32),
            pltpu.VMEM((CHUNK, EMBED_DIM), jnp.float32),
            pltpu.SemaphoreType.DMA,
        ],
        compiler_params=pltpu.CompilerParams(use_tc_tiling_on_sc=False),
    )
    def k(idx_hbm, table_hbm, out_hbm, idx_v, rows_v, sem):
        wid = lax.axis_index("s") * NUM_CORES + lax.axis_index("c")
        base = wid * b_per_w

        def chunk_body(g, carry):
            off = pl.multiple_of(base + g * CHUNK, CHUNK)
            row_off = pl.multiple_of(off // IDX_SUB, SUBS_PER_CHUNK)
            pltpu.sync_copy(idx_hbm.at[pl.ds(row_off, SUBS_PER_CHUNK)], idx_v)
            copies = []
            for j in range(SUBS_PER_CHUNK):
                copies.append(
                    pltpu.async_copy(
                        table_hbm.at[idx_v.at[j]],
                        rows_v.at[pl.ds(j * IDX_SUB, IDX_SUB)],
                        sem,
                    )
                )
            for c in copies:
                c.wait()

            def scale_body(i, c2):
                for j in range(EMBED_DIM // 16):
                    sl = pl.ds(j * 16, 16)
                    rows_v[i, sl] = rows_v[i, sl] * SCALE
                return c2

            lax.fori_loop(0, CHUNK, scale_body, 0, unroll=4)
            pltpu.sync_copy(rows_v, out_hbm.at[pl.ds(off, CHUNK)])
            return carry

        lax.fori_loop(0, n_chunks, chunk_body, 0)

    return k


def kernel(x, table):
    n_rows = x.size
    idx2d = x.reshape(n_rows // IDX_SUB, IDX_SUB).astype(jnp.int32)
    out = _make_kernel(n_rows)(idx2d, table)
    return out.reshape(x.shape + (EMBED_DIM,))
